# Initial kernel scaffold; baseline (speedup 1.0000x reference)
import jax
import jax.numpy as jnp
from jax.experimental import pallas as pl


def kernel(tokens, table):
    raise NotImplementedError("write your pallas kernel here")


# speedup vs baseline: 31.2706x; 31.2706x over previous
import functools

import jax
import jax.numpy as jnp
from jax.experimental import pallas as pl
from jax.experimental.pallas import tpu as pltpu

VOCAB = 30
OUT_LEN = 256
EMB_DIM = OUT_LEN * VOCAB
SEQ = 50
BLOCK_B = 256


def _body(tok_ref, table_ref, out_ref):
    tok = tok_ref[...]
    vocab_ids = jax.lax.broadcasted_iota(jnp.int32, (1, 1, VOCAB), 2)
    onehot = (tok[:, :, None] == vocab_ids).astype(jnp.float32)
    counts = jnp.sum(onehot, axis=1)
    out_ref[...] = jnp.dot(counts, table_ref[...],
                           preferred_element_type=jnp.float32)


@jax.jit
def kernel(tokens, table):
    batch = tokens.shape[0]
    tokens = tokens.astype(jnp.int32)
    grid = (batch // BLOCK_B,)
    out = pl.pallas_call(
        _body,
        grid=grid,
        in_specs=[
            pl.BlockSpec((BLOCK_B, SEQ), lambda i: (i, 0)),
            pl.BlockSpec((VOCAB, EMB_DIM), lambda i: (0, 0)),
        ],
        out_specs=pl.BlockSpec((BLOCK_B, EMB_DIM), lambda i: (i, 0)),
        out_shape=jax.ShapeDtypeStruct((batch, EMB_DIM), jnp.float32),
        compiler_params=pltpu.CompilerParams(
            dimension_semantics=("parallel",),
        ),
    )(tokens, table)
    return out.reshape(batch, OUT_LEN, VOCAB)
